# baseline (speedup 1.0000x reference)
import jax
import jax.numpy as jnp
from jax.experimental import pallas as pl
from jax.experimental.pallas import tpu as pltpu

_G = 256
_D = 128
_GSD = 128
_H = 2 * _GSD
_TILE = 2048


def _fused(x_ref, idx_ref, w1_ref, b1_ref, w2_ref, b2_ref, out_ref, acc_ref,
           *, n_rows):
    i = pl.program_id(0)
    nsteps = pl.num_programs(0)

    @pl.when(i == 0)
    def _init():
        acc_ref[...] = jnp.zeros_like(acc_ref)

    x = x_ref[...]
    h = jax.lax.dot_general(x, w1_ref[...], (((1,), (1,)), ((), ())),
                            preferred_element_type=jnp.float32)
    h = h + b1_ref[...]
    g = h[:, _GSD:] * jax.nn.sigmoid(h[:, :_GSD])

    row = i * _TILE + jax.lax.broadcasted_iota(jnp.int32, (_TILE, 1), 0)
    g = jnp.where(row < n_rows, g, 0.0)

    idx = idx_ref[0, 0, :]
    onehot = (jax.lax.broadcasted_iota(jnp.int32, (_G, _TILE), 0)
              == idx[None, :]).astype(jnp.bfloat16)
    acc_ref[...] += jax.lax.dot_general(onehot, g.astype(jnp.bfloat16),
                                        (((1,), (0,)), ((), ())),
                                        preferred_element_type=jnp.float32)

    @pl.when(i == nsteps - 1)
    def _finish():
        out = jax.lax.dot_general(acc_ref[...], w2_ref[...],
                                  (((1,), (1,)), ((), ())),
                                  preferred_element_type=jnp.float32)
        out_ref[...] = out + b2_ref[...]


def kernel(node_states, graph_idx, n_graphs, W1, b1, W2, b2):
    n = node_states.shape[0]
    nsteps = pl.cdiv(n, _TILE)
    npad = nsteps * _TILE
    idx = jnp.minimum(graph_idx.astype(jnp.int32), _G - 1)
    idx = jnp.pad(idx, (0, npad - n), constant_values=_G)
    idx3 = idx.reshape(nsteps, 1, _TILE)

    import functools
    out = pl.pallas_call(
        functools.partial(_fused, n_rows=n),
        grid=(nsteps,),
        in_specs=[
            pl.BlockSpec((_TILE, _D), lambda i: (i, 0)),
            pl.BlockSpec((1, 1, _TILE), lambda i: (i, 0, 0)),
            pl.BlockSpec((_H, _D), lambda i: (0, 0)),
            pl.BlockSpec((1, _H), lambda i: (0, 0)),
            pl.BlockSpec((_GSD, _GSD), lambda i: (0, 0)),
            pl.BlockSpec((1, _GSD), lambda i: (0, 0)),
        ],
        out_specs=pl.BlockSpec((_G, _GSD), lambda i: (0, 0)),
        out_shape=jax.ShapeDtypeStruct((_G, _GSD), jnp.float32),
        scratch_shapes=[pltpu.VMEM((_G, _GSD), jnp.float32)],
        compiler_params=pltpu.CompilerParams(
            dimension_semantics=("arbitrary",)),
    )(node_states, idx3, W1, b1.reshape(1, _H), W2, b2.reshape(1, _GSD))
    return out


# speedup vs baseline: 1.4721x; 1.4721x over previous
import jax
import jax.numpy as jnp
from jax.experimental import pallas as pl
from jax.experimental.pallas import tpu as pltpu

_G = 256
_D = 128
_GSD = 128
_H = 2 * _GSD
_TILE = 8192


def _fused(x_ref, idx_ref, w1_ref, b1_ref, w2_ref, b2_ref, out_ref, acc_ref,
           *, n_rows):
    i = pl.program_id(0)
    nsteps = pl.num_programs(0)

    @pl.when(i == 0)
    def _init():
        acc_ref[...] = jnp.zeros_like(acc_ref)

    x = x_ref[...]
    h = jax.lax.dot_general(x, w1_ref[...], (((1,), (1,)), ((), ())),
                            preferred_element_type=jnp.float32)
    h = h + b1_ref[...]
    g = h[:, _GSD:] * jax.nn.sigmoid(h[:, :_GSD])

    row = i * _TILE + jax.lax.broadcasted_iota(jnp.int32, (_TILE, 1), 0)
    g = jnp.where(row < n_rows, g, 0.0)

    idx = idx_ref[0, 0, :]
    onehot = (jax.lax.broadcasted_iota(jnp.int32, (_G, _TILE), 0)
              == idx[None, :]).astype(jnp.bfloat16)
    acc_ref[...] += jax.lax.dot_general(onehot, g.astype(jnp.bfloat16),
                                        (((1,), (0,)), ((), ())),
                                        preferred_element_type=jnp.float32)

    @pl.when(i == nsteps - 1)
    def _finish():
        out = jax.lax.dot_general(acc_ref[...], w2_ref[...],
                                  (((1,), (1,)), ((), ())),
                                  preferred_element_type=jnp.float32)
        out_ref[...] = out + b2_ref[...]


def kernel(node_states, graph_idx, n_graphs, W1, b1, W2, b2):
    n = node_states.shape[0]
    nsteps = pl.cdiv(n, _TILE)
    npad = nsteps * _TILE
    idx = jnp.minimum(graph_idx.astype(jnp.int32), _G - 1)
    idx = jnp.pad(idx, (0, npad - n), constant_values=_G)
    idx3 = idx.reshape(nsteps, 1, _TILE)

    import functools
    out = pl.pallas_call(
        functools.partial(_fused, n_rows=n),
        grid=(nsteps,),
        in_specs=[
            pl.BlockSpec((_TILE, _D), lambda i: (i, 0)),
            pl.BlockSpec((1, 1, _TILE), lambda i: (i, 0, 0)),
            pl.BlockSpec((_H, _D), lambda i: (0, 0)),
            pl.BlockSpec((1, _H), lambda i: (0, 0)),
            pl.BlockSpec((_GSD, _GSD), lambda i: (0, 0)),
            pl.BlockSpec((1, _GSD), lambda i: (0, 0)),
        ],
        out_specs=pl.BlockSpec((_G, _GSD), lambda i: (0, 0)),
        out_shape=jax.ShapeDtypeStruct((_G, _GSD), jnp.float32),
        scratch_shapes=[pltpu.VMEM((_G, _GSD), jnp.float32)],
        compiler_params=pltpu.CompilerParams(
            dimension_semantics=("arbitrary",)),
    )(node_states, idx3, W1, b1.reshape(1, _H), W2, b2.reshape(1, _GSD))
    return out
